# Initial kernel scaffold; baseline (speedup 1.0000x reference)
import jax
import jax.numpy as jnp
from jax.experimental import pallas as pl


def kernel(pred, targets, fc_w1, fc_w2):
    raise NotImplementedError("write your pallas kernel here")


# speedup vs baseline: 1.3512x; 1.3512x over previous
import functools

import jax
import jax.numpy as jnp
from jax import lax
from jax.experimental import pallas as pl
from jax.experimental.pallas import tpu as pltpu
from jax.experimental.pallas import tpu_sc as plsc

B = 16384
NC = 1000
D = 128
S = 0.1 / NC


NUM_WORKERS = 32
BPW = B // NUM_WORKERS
LANES = 16
CHUNK = 128
NCHUNK = BPW // CHUNK


def _gather_body(pred_hbm, tgt_hbm, out_hbm, tvec, idxv, rows, sem):
    wid = lax.axis_index("s") * 2 + lax.axis_index("c")
    base = wid * BPW
    pltpu.sync_copy(tgt_hbm.at[pl.ds(base, BPW)], tvec)

    def compute_idx(k, carry):
        t = tvec[pl.ds(k * LANES, LANES)]
        row = lax.iota(jnp.int32, LANES) + (base + k * LANES)
        idxv[pl.ds(k * LANES, LANES)] = row * NC + t
        return carry

    lax.fori_loop(0, BPW // LANES, compute_idx, 0)

    copies = [
        pltpu.async_copy(
            pred_hbm.at[idxv.at[pl.ds(j * CHUNK, CHUNK)]],
            rows.at[pl.ds(j * CHUNK, CHUNK)],
            sem,
        )
        for j in range(NCHUNK)
    ]
    for cp in copies:
        cp.wait()
    pltpu.sync_copy(rows, out_hbm.at[pl.ds(base, BPW)])


@functools.cache
def _make_gather_pt():
    return pl.kernel(
        _gather_body,
        mesh=plsc.VectorSubcoreMesh(core_axis_name="c", subcore_axis_name="s"),
        out_type=jax.ShapeDtypeStruct((B,), jnp.float32),
        scratch_types=[
            pltpu.VMEM((BPW,), jnp.int32),
            pltpu.VMEM((BPW,), jnp.int32),
            pltpu.VMEM((BPW,), jnp.float32),
            pltpu.SemaphoreType.DMA,
        ],
    )


ROWS_PER_BLOCK = 1024
NBLK = B // ROWS_PER_BLOCK


def _loss_body(pred_ref, pt_ref, w1_ref, w2_ref, out_ref, acc_ref):
    i = pl.program_id(0)
    p = pred_ref[...]
    l1 = jnp.maximum(jnp.log(p), -100.0)
    l2 = jnp.maximum(jnp.log(1.0 - p), -100.0)
    s1 = jnp.sum(l1)
    s2 = jnp.sum(l2)

    @pl.when(i == 0)
    def _init():
        acc_ref[0] = s1
        acc_ref[1] = s2

    @pl.when(i > 0)
    def _acc():
        acc_ref[0] += s1
        acc_ref[1] += s2

    @pl.when(i == NBLK - 1)
    def _finish():
        pt = pt_ref[...]
        l1t = jnp.maximum(jnp.log(pt), -100.0)
        l2t = jnp.maximum(jnp.log(1.0 - pt), -100.0)
        corr = jnp.sum(l1t - l2t)
        w1 = w1_ref[...]
        w2 = w2_ref[...]
        num = jnp.sum(w1 * w2, axis=1, keepdims=True)
        n1 = jnp.sqrt(jnp.sum(w1 * w1, axis=1, keepdims=True))
        n2 = jnp.sqrt(jnp.sum(w2 * w2, axis=1, keepdims=True))
        sim = jnp.abs(num / jnp.maximum(n1 * n2, 1e-8))
        sim = jnp.clip(sim, 1e-4, 0.9999)
        wsim = -jnp.sum(jnp.log(1.0 - sim)) * (1.0 / NC)
        bce = -(S * acc_ref[0] + (1.0 - S) * acc_ref[1] + 0.9 * corr) / (
            B * NC
        )
        out_ref[0, 0] = bce + 100.0 * wsim


_loss = pl.pallas_call(
    _loss_body,
    grid=(NBLK,),
    in_specs=[
        pl.BlockSpec((ROWS_PER_BLOCK, NC), lambda i: (i, 0)),
        pl.BlockSpec((128, 128), lambda i: (0, 0)),
        pl.BlockSpec((NC, D), lambda i: (0, 0)),
        pl.BlockSpec((NC, D), lambda i: (0, 0)),
    ],
    out_specs=pl.BlockSpec(memory_space=pltpu.SMEM),
    out_shape=jax.ShapeDtypeStruct((1, 1), jnp.float32),
    scratch_shapes=[pltpu.SMEM((2,), jnp.float32)],
    compiler_params=pltpu.CompilerParams(dimension_semantics=("arbitrary",)),
)


def kernel(pred, targets, fc_w1, fc_w2):
    pt = _make_gather_pt()(pred.reshape(-1), targets)
    out = _loss(pred, pt.reshape(128, 128), fc_w1, fc_w2)
    return out[0, 0]


# speedup vs baseline: 2.6744x; 1.9792x over previous
import functools

import jax
import jax.numpy as jnp
from jax import lax
from jax.experimental import pallas as pl
from jax.experimental.pallas import tpu as pltpu

B = 16384
NC = 1000
D = 128
S = 0.1 / NC

ROWS_PER_BLOCK = 1024
NBLK = B // ROWS_PER_BLOCK


def _loss_body(pred_ref, tgt_ref, w1_ref, w2_ref, out_ref, acc_ref):
    i = pl.program_id(0)
    p = pred_ref[...]
    l1 = jnp.maximum(jnp.log(p), -100.0)
    l2 = jnp.maximum(jnp.log(1.0 - p), -100.0)
    diff = l1 - l2
    t = tgt_ref[0, 0, :]
    cols = lax.broadcasted_iota(jnp.int32, (ROWS_PER_BLOCK, NC), 1)
    hit = cols == t[:, None]
    s1 = jnp.sum(l1)
    s2 = jnp.sum(l2)
    sc = jnp.sum(jnp.where(hit, diff, 0.0))

    @pl.when(i == 0)
    def _init():
        acc_ref[0] = s1
        acc_ref[1] = s2
        acc_ref[2] = sc

    @pl.when(i > 0)
    def _acc():
        acc_ref[0] += s1
        acc_ref[1] += s2
        acc_ref[2] += sc

    @pl.when(i == NBLK - 1)
    def _finish():
        w1 = w1_ref[...]
        w2 = w2_ref[...]
        num = jnp.sum(w1 * w2, axis=1, keepdims=True)
        n1 = jnp.sqrt(jnp.sum(w1 * w1, axis=1, keepdims=True))
        n2 = jnp.sqrt(jnp.sum(w2 * w2, axis=1, keepdims=True))
        sim = jnp.abs(num / jnp.maximum(n1 * n2, 1e-8))
        sim = jnp.clip(sim, 1e-4, 0.9999)
        wsim = -jnp.sum(jnp.log(1.0 - sim)) * (1.0 / NC)
        bce = -(S * acc_ref[0] + (1.0 - S) * acc_ref[1] + 0.9 * acc_ref[2]) / (
            B * NC
        )
        out_ref[0, 0] = bce + 100.0 * wsim


_loss = pl.pallas_call(
    _loss_body,
    grid=(NBLK,),
    in_specs=[
        pl.BlockSpec((ROWS_PER_BLOCK, NC), lambda i: (i, 0)),
        pl.BlockSpec((1, 1, ROWS_PER_BLOCK), lambda i: (i, 0, 0)),
        pl.BlockSpec((NC, D), lambda i: (0, 0)),
        pl.BlockSpec((NC, D), lambda i: (0, 0)),
    ],
    out_specs=pl.BlockSpec(memory_space=pltpu.SMEM),
    out_shape=jax.ShapeDtypeStruct((1, 1), jnp.float32),
    scratch_shapes=[pltpu.SMEM((3,), jnp.float32)],
    compiler_params=pltpu.CompilerParams(dimension_semantics=("arbitrary",)),
)


def kernel(pred, targets, fc_w1, fc_w2):
    tgt3 = targets.reshape(NBLK, 1, ROWS_PER_BLOCK)
    out = _loss(pred, tgt3, fc_w1, fc_w2)
    return out[0, 0]


# speedup vs baseline: 5.1999x; 1.9443x over previous
import functools

import jax
import jax.numpy as jnp
from jax import lax
from jax.experimental import pallas as pl
from jax.experimental.pallas import tpu as pltpu

B = 16384
NC = 1000
D = 128
S = 0.1 / NC

COLS_PER_BLOCK = 1024
NBLK = B // COLS_PER_BLOCK


def _loss_body(pred_ref, tgt_ref, w1_ref, w2_ref, out_ref, acc_ref):
    i = pl.program_id(0)
    p = pred_ref[...]
    l1 = jnp.maximum(jnp.log(p), -100.0)
    l2 = jnp.maximum(jnp.log(1.0 - p), -100.0)
    diff = l1 - l2
    t = tgt_ref[0, 0, :]
    classes = lax.broadcasted_iota(jnp.int32, (NC, COLS_PER_BLOCK), 0)
    hit = classes == t[None, :]
    s1 = jnp.sum(l1)
    s2 = jnp.sum(l2)
    sc = jnp.sum(jnp.where(hit, diff, 0.0))

    @pl.when(i == 0)
    def _init():
        acc_ref[0] = s1
        acc_ref[1] = s2
        acc_ref[2] = sc

    @pl.when(i > 0)
    def _acc():
        acc_ref[0] += s1
        acc_ref[1] += s2
        acc_ref[2] += sc

    @pl.when(i == NBLK - 1)
    def _finish():
        w1 = w1_ref[...]
        w2 = w2_ref[...]
        num = jnp.sum(w1 * w2, axis=1, keepdims=True)
        n1 = jnp.sqrt(jnp.sum(w1 * w1, axis=1, keepdims=True))
        n2 = jnp.sqrt(jnp.sum(w2 * w2, axis=1, keepdims=True))
        sim = jnp.abs(num / jnp.maximum(n1 * n2, 1e-8))
        sim = jnp.clip(sim, 1e-4, 0.9999)
        wsim = -jnp.sum(jnp.log(1.0 - sim)) * (1.0 / NC)
        bce = -(S * acc_ref[0] + (1.0 - S) * acc_ref[1] + 0.9 * acc_ref[2]) / (
            B * NC
        )
        out_ref[0, 0] = bce + 100.0 * wsim


_loss = pl.pallas_call(
    _loss_body,
    grid=(NBLK,),
    in_specs=[
        pl.BlockSpec((NC, COLS_PER_BLOCK), lambda i: (0, i)),
        pl.BlockSpec((1, 1, COLS_PER_BLOCK), lambda i: (i, 0, 0)),
        pl.BlockSpec((NC, D), lambda i: (0, 0)),
        pl.BlockSpec((NC, D), lambda i: (0, 0)),
    ],
    out_specs=pl.BlockSpec(memory_space=pltpu.SMEM),
    out_shape=jax.ShapeDtypeStruct((1, 1), jnp.float32),
    scratch_shapes=[pltpu.SMEM((3,), jnp.float32)],
    compiler_params=pltpu.CompilerParams(dimension_semantics=("arbitrary",)),
)


def kernel(pred, targets, fc_w1, fc_w2):
    tgt3 = targets.reshape(NBLK, 1, COLS_PER_BLOCK)
    out = _loss(pred.T, tgt3, fc_w1, fc_w2)
    return out[0, 0]
